# baseline (speedup 1.0000x reference)
import jax
import jax.numpy as jnp
from jax import lax
from jax.experimental import pallas as pl
from jax.experimental.pallas import tpu as pltpu
from jax.experimental.pallas import tpu_sc as plsc

B = 16384
COLS = 20
NC = 2
NS = 16
LANES = 16
NW = NC * NS
CHUNK = B // NW
GROUPS = CHUNK // LANES
UNROLL = 2


def _body(xt_hbm, emb_hbm, out_hbm, xbuf, embbuf, obuf):
    cid = lax.axis_index("c")
    sid = lax.axis_index("s")
    wid = sid * NC + cid
    base = wid * CHUNK

    pltpu.sync_copy(xt_hbm.at[:, pl.ds(base, CHUNK)], xbuf)
    pltpu.sync_copy(emb_hbm, embbuf)

    @plsc.parallel_loop(0, GROUPS, step=1, unroll=UNROLL)
    def _loop(i):
        o = i * LANES
        cols = [xbuf[j, pl.ds(o, LANES)] for j in range(COLS)]
        obj = cols[15:20]
        for g in range(3):
            acc = None
            for j in range(5):
                idx = cols[5 * g + j].astype(jnp.int32)
                w = plsc.load_gather(embbuf, [idx])
                t = w * obj[j]
                acc = t if acc is None else acc + t
            obuf[g, pl.ds(o, LANES)] = acc

    pltpu.sync_copy(obuf, out_hbm.at[:, pl.ds(base, CHUNK)])


@jax.jit
def kernel(x, level_embedding):
    xt = x.T
    emb = level_embedding.reshape(5)
    mesh = plsc.VectorSubcoreMesh(
        core_axis_name="c", subcore_axis_name="s",
        num_cores=NC, num_subcores=NS,
    )
    run = pl.kernel(
        _body,
        out_type=jax.ShapeDtypeStruct((3, B), jnp.float32),
        mesh=mesh,
        scratch_types=[
            pltpu.VMEM((COLS, CHUNK), jnp.float32),
            pltpu.VMEM((5,), jnp.float32),
            pltpu.VMEM((3, CHUNK), jnp.float32),
        ],
        compiler_params=pltpu.CompilerParams(
            needs_layout_passes=False,
            use_tc_tiling_on_sc=True,
            disable_bounds_checks=True,
        ),
    )
    return run(xt, emb).T


# speedup vs baseline: 1.0010x; 1.0010x over previous
import jax
import jax.numpy as jnp
from jax import lax
from jax.experimental import pallas as pl
from jax.experimental.pallas import tpu as pltpu
from jax.experimental.pallas import tpu_sc as plsc

B = 16384
COLS = 20
NC = 2
NS = 16
LANES = 16
NW = NC * NS
CHUNK = B // NW
HALF = CHUNK // 2
GROUPS = CHUNK // LANES
UNROLL = 2


def _body(xt_hbm, emb_hbm, out_hbm, xbuf, embbuf, obuf, sem0, sem1):
    cid = lax.axis_index("c")
    sid = lax.axis_index("s")
    wid = sid * NC + cid
    base = wid * CHUNK

    c0 = pltpu.async_copy(
        xt_hbm.at[:, pl.ds(base, HALF)], xbuf.at[:, pl.ds(0, HALF)], sem0)
    c1 = pltpu.async_copy(
        xt_hbm.at[:, pl.ds(base + HALF, HALF)], xbuf.at[:, pl.ds(HALF, HALF)], sem1)
    pltpu.sync_copy(emb_hbm, embbuf)

    def make_loop(lo, hi):
        @plsc.parallel_loop(lo, hi, step=1, unroll=UNROLL)
        def _loop(i):
            o = i * LANES
            cols = [xbuf[j, pl.ds(o, LANES)] for j in range(COLS)]
            obj = cols[15:20]
            for g in range(3):
                acc = None
                for j in range(5):
                    idx = cols[5 * g + j].astype(jnp.int32)
                    w = plsc.load_gather(embbuf, [idx])
                    t = w * obj[j]
                    acc = t if acc is None else acc + t
                obuf[g, pl.ds(o, LANES)] = acc

    c0.wait()
    make_loop(0, GROUPS // 2)
    c1.wait()
    make_loop(GROUPS // 2, GROUPS)

    pltpu.sync_copy(obuf, out_hbm.at[:, pl.ds(base, CHUNK)])


@jax.jit
def kernel(x, level_embedding):
    xt = x.T
    emb = level_embedding.reshape(5)
    mesh = plsc.VectorSubcoreMesh(
        core_axis_name="c", subcore_axis_name="s",
        num_cores=NC, num_subcores=NS,
    )
    run = pl.kernel(
        _body,
        out_type=jax.ShapeDtypeStruct((3, B), jnp.float32),
        mesh=mesh,
        scratch_types=[
            pltpu.VMEM((COLS, CHUNK), jnp.float32),
            pltpu.VMEM((5,), jnp.float32),
            pltpu.VMEM((3, CHUNK), jnp.float32),
            pltpu.SemaphoreType.DMA,
            pltpu.SemaphoreType.DMA,
        ],
        compiler_params=pltpu.CompilerParams(
            needs_layout_passes=False,
            use_tc_tiling_on_sc=True,
            disable_bounds_checks=True,
        ),
    )
    return run(xt, emb).T
